# baseline (speedup 1.0000x reference)
import functools

import jax
import jax.numpy as jnp
from jax import lax
from jax.experimental import pallas as pl
from jax.experimental.pallas import tpu as pltpu
from jax.experimental.pallas import tpu_sc as plsc

_B, _S, _D = 4, 4096, 2048
_SURV = _S - (_S + 2) // 3
_TOT = _B * _SURV
_NC, _NS = 2, 16
_NW = _NC * _NS
_CH = 24
_FULL = 14
_IDXCAP = 352


@functools.partial(
    pl.kernel,
    mesh=plsc.VectorSubcoreMesh(core_axis_name="c", subcore_axis_name="s"),
    compiler_params=pltpu.CompilerParams(use_tc_tiling_on_sc=False),
    out_type=jax.ShapeDtypeStruct((_TOT, _D), jnp.float32),
    scratch_types=[
        pltpu.VMEM((_IDXCAP,), jnp.int32),
        pltpu.VMEM((_CH, _D), jnp.float32),
        pltpu.VMEM((_CH, _D), jnp.float32),
        pltpu.VMEM((8, _D), jnp.float32),
        pltpu.SemaphoreType.DMA,
        pltpu.SemaphoreType.DMA,
        pltpu.SemaphoreType.DMA,
    ],
)
def _prune(table, out, idx_v, buf0, buf1, tb8, g0, g1, tsem):
    wid = lax.axis_index("s") * _NC + lax.axis_index("c")
    base = wid * 336 + 8 * jnp.minimum(wid, 21)
    lanes = lax.iota(jnp.int32, 16)
    for i in range(_IDXCAP // 16):
        r = base + i * 16 + lanes
        bsel = lax.div(r, jnp.int32(_SURV))
        j = r - bsel * _SURV
        src = bsel * _S + j + (j >> 1) + 1
        idx_v[pl.ds(i * 16, 16)] = jnp.minimum(src, _B * _S - 1)

    bufs = (buf0, buf1)
    gsems = (g0, g1)
    copies = [
        pltpu.async_copy(table.at[idx_v.at[pl.ds(0, _CH)]], buf0, g0),
        pltpu.async_copy(table.at[idx_v.at[pl.ds(_CH, _CH)]], buf1, g1),
    ]
    for t in range(_FULL):
        s = t % 2
        copies[s].wait()
        pltpu.sync_copy(bufs[s], out.at[pl.ds(base + t * _CH, _CH)])
        nxt = t + 2
        if nxt < _FULL:
            copies[s] = pltpu.async_copy(
                table.at[idx_v.at[pl.ds(nxt * _CH, _CH)]], bufs[s], gsems[s])

    tail = _FULL * _CH

    @pl.when(wid < 21)
    def _tail8():
        pltpu.async_copy(table.at[idx_v.at[pl.ds(tail, 8)]], tb8, tsem).wait()
        pltpu.sync_copy(tb8, out.at[pl.ds(base + tail, 8)])


def kernel(hidden_states):
    table = hidden_states.reshape(_B * _S, _D)
    flat = _prune(table)
    return flat.reshape(_B, _SURV, _D)


# speedup vs baseline: 3.0607x; 3.0607x over previous
import functools

import jax
import jax.numpy as jnp
from jax import lax
from jax.experimental import pallas as pl
from jax.experimental.pallas import tpu as pltpu
from jax.experimental.pallas import tpu_sc as plsc

_B, _S, _D = 4, 4096, 2048
_SURV = _S - (_S + 2) // 3
_TOT = _B * _SURV
_NC, _NS = 2, 16
_NW = _NC * _NS
_CH = 24
_FULL = 14
_IDXCAP = 352


@functools.partial(
    pl.kernel,
    mesh=plsc.VectorSubcoreMesh(core_axis_name="c", subcore_axis_name="s"),
    out_type=jax.ShapeDtypeStruct((_TOT, _D), jnp.float32),
    scratch_types=[
        pltpu.VMEM((_IDXCAP,), jnp.int32),
        pltpu.VMEM((_CH, _D), jnp.float32),
        pltpu.VMEM((_CH, _D), jnp.float32),
        pltpu.VMEM((8, _D), jnp.float32),
        pltpu.SemaphoreType.DMA,
        pltpu.SemaphoreType.DMA,
        pltpu.SemaphoreType.DMA,
    ],
)
def _prune(table, out, idx_v, buf0, buf1, tb8, g0, g1, tsem):
    wid = lax.axis_index("s") * _NC + lax.axis_index("c")
    base = wid * 336 + 8 * jnp.minimum(wid, 21)
    lanes = lax.iota(jnp.int32, 16)
    for i in range(_IDXCAP // 16):
        r = base + i * 16 + lanes
        bsel = lax.div(r, jnp.int32(_SURV))
        j = r - bsel * _SURV
        src = bsel * _S + j + (j >> 1) + 1
        idx_v[pl.ds(i * 16, 16)] = jnp.minimum(src, _B * _S - 1)

    bufs = (buf0, buf1)
    gsems = (g0, g1)
    copies = [
        pltpu.async_copy(table.at[idx_v.at[pl.ds(0, _CH)]], buf0, g0),
        pltpu.async_copy(table.at[idx_v.at[pl.ds(_CH, _CH)]], buf1, g1),
    ]
    for t in range(_FULL):
        s = t % 2
        copies[s].wait()
        pltpu.sync_copy(bufs[s], out.at[pl.ds(base + t * _CH, _CH)])
        nxt = t + 2
        if nxt < _FULL:
            copies[s] = pltpu.async_copy(
                table.at[idx_v.at[pl.ds(nxt * _CH, _CH)]], bufs[s], gsems[s])

    tail = _FULL * _CH

    @pl.when(wid < 21)
    def _tail8():
        pltpu.async_copy(table.at[idx_v.at[pl.ds(tail, 8)]], tb8, tsem).wait()
        pltpu.sync_copy(tb8, out.at[pl.ds(base + tail, 8)])


def kernel(hidden_states):
    table = hidden_states.reshape(_B * _S, _D)
    flat = _prune(table)
    return flat.reshape(_B, _SURV, _D)


# speedup vs baseline: 3.7637x; 1.2297x over previous
import functools

import jax
import jax.numpy as jnp
from jax import lax
from jax.experimental import pallas as pl
from jax.experimental.pallas import tpu as pltpu
from jax.experimental.pallas import tpu_sc as plsc

_B, _S, _D = 4, 4096, 2048
_SURV = _S - (_S + 2) // 3
_NC, _NS = 2, 16
_CH = 24
_FULL = 14
_IDXCAP = 352


@functools.partial(
    pl.kernel,
    mesh=plsc.VectorSubcoreMesh(core_axis_name="c", subcore_axis_name="s"),
    out_type=jax.ShapeDtypeStruct((_B, _SURV, _D), jnp.float32),
    scratch_types=[
        pltpu.VMEM((_IDXCAP,), jnp.int32),
        pltpu.VMEM((_CH, _D), jnp.float32),
        pltpu.VMEM((_CH, _D), jnp.float32),
        pltpu.VMEM((8, _D), jnp.float32),
        pltpu.VMEM((2, _D), jnp.float32),
        pltpu.SemaphoreType.DMA,
        pltpu.SemaphoreType.DMA,
        pltpu.SemaphoreType.DMA,
    ],
)
def _prune(table, out, idx_v, buf0, buf1, tb8, tb2, g0, g1, tsem):
    wid = lax.axis_index("s") * _NC + lax.axis_index("c")
    b = wid // 8
    o = wid % 8
    base = o * 336 + 8 * jnp.minimum(o, 5)
    lanes = lax.iota(jnp.int32, 16)
    for i in range(_IDXCAP // 16):
        j = base + i * 16 + lanes
        src = b * _S + j + (j >> 1) + 1
        idx_v[pl.ds(i * 16, 16)] = jnp.minimum(src, _B * _S - 1)

    bufs = (buf0, buf1)
    gsems = (g0, g1)
    copies = [
        pltpu.async_copy(table.at[idx_v.at[pl.ds(0, _CH)]], buf0, g0),
        pltpu.async_copy(table.at[idx_v.at[pl.ds(_CH, _CH)]], buf1, g1),
    ]
    for t in range(_FULL):
        s = t % 2
        copies[s].wait()
        pltpu.sync_copy(bufs[s], out.at[b, pl.ds(base + t * _CH, _CH)])
        nxt = t + 2
        if nxt < _FULL:
            copies[s] = pltpu.async_copy(
                table.at[idx_v.at[pl.ds(nxt * _CH, _CH)]], bufs[s], gsems[s])

    tail = _FULL * _CH

    @pl.when(o < 5)
    def _tail8():
        pltpu.async_copy(table.at[idx_v.at[pl.ds(tail, 8)]], tb8, tsem).wait()
        pltpu.sync_copy(tb8, out.at[b, pl.ds(base + tail, 8)])

    @pl.when(o == 7)
    def _tail2():
        pltpu.async_copy(table.at[idx_v.at[pl.ds(tail, 2)]], tb2, tsem).wait()
        pltpu.sync_copy(tb2, out.at[b, pl.ds(2728, 2)])


def kernel(hidden_states):
    table = hidden_states.reshape(_B * _S, _D)
    return _prune(table)
